# baseline (speedup 1.0000x reference)
import functools
import jax
import jax.numpy as jnp
from jax import lax
from jax.experimental import pallas as pl
from jax.experimental.pallas import tpu as pltpu
from jax.experimental.pallas import tpu_sc as plsc

_NC, _NS, _L = 2, 16, 16
_NW = _NC * _NS


def _sc_body(num_patches, dim, batch, ppw, cp,
             x_hbm, table_hbm, pos_hbm, out_hbm,
             idx_v, tbuf, xbuf, in_sems, out_sems, gat_sems):
    wid = lax.axis_index("s") * _NC + lax.axis_index("c")
    base = wid * ppw
    pltpu.sync_copy(pos_hbm.at[pl.ds(base, ppw)], idx_v)

    n_chunks = ppw // cp

    def gather_desc(c):
        s = c % 2
        return pltpu.make_async_copy(
            table_hbm.at[idx_v.at[pl.ds(c * cp, cp)]], tbuf.at[s],
            gat_sems.at[s])

    def xin_desc(c, b):
        s = c % 2
        return pltpu.make_async_copy(
            x_hbm.at[b, pl.ds(base + c * cp, cp)], xbuf.at[s, b],
            in_sems.at[s, b])

    def xout_desc(c, b):
        s = c % 2
        return pltpu.make_async_copy(
            xbuf.at[s, b], out_hbm.at[b, pl.ds(base + c * cp, cp)],
            out_sems.at[s, b])

    gather_desc(0).start()
    for b in range(batch):
        xin_desc(0, b).start()

    def chunk_step(c, _):
        s = c % 2

        @pl.when(c >= 1)
        def _wait_prev_outs():
            for b in range(batch):
                xout_desc(c - 1, b).wait()

        @pl.when(c + 1 < n_chunks)
        def _prefetch_next():
            gather_desc(c + 1).start()
            for b in range(batch):
                xin_desc(c + 1, b).start()

        gather_desc(c).wait()

        for b in range(batch):
            xin_desc(c, b).wait()

            def row_add(r, carry, b=b):
                for k in range(dim // _L):
                    sl = pl.ds(k * _L, _L)
                    plsc.addupdate(xbuf.at[s, b, r, sl], tbuf[s, r, sl])
                return carry

            lax.fori_loop(0, cp, row_add, None)
            xout_desc(c, b).start()
        return _

    lax.fori_loop(0, n_chunks, chunk_step, None)
    for b in range(batch):
        xout_desc(n_chunks - 1, b).wait()


def sc_kernel(encoded_patches, position_embedding, positions):
    batch, num_patches, dim = encoded_patches.shape
    ppw = num_patches // _NW
    cp = 8

    mesh = plsc.VectorSubcoreMesh(core_axis_name="c", subcore_axis_name="s")
    body = functools.partial(_sc_body, num_patches, dim, batch, ppw, cp)
    return pl.kernel(
        body,
        out_type=jax.ShapeDtypeStruct(encoded_patches.shape, encoded_patches.dtype),
        mesh=mesh,
        scratch_types=[
            pltpu.VMEM((ppw,), jnp.int32),
            pltpu.VMEM((2, cp, dim), jnp.float32),
            pltpu.VMEM((2, batch, cp, dim), jnp.float32),
            pltpu.SemaphoreType.DMA((2, batch)),
            pltpu.SemaphoreType.DMA((2, batch)),
            pltpu.SemaphoreType.DMA((2,)),
        ],
    )(encoded_patches, position_embedding, positions)


def kernel(encoded_patches, position_embedding, positions):
    return sc_kernel(encoded_patches, position_embedding, positions)
